# baseline (speedup 1.0000x reference)
import functools

import jax
import jax.numpy as jnp
from jax import lax
from jax.experimental import pallas as pl
from jax.experimental.pallas import tpu as pltpu
from jax.experimental.pallas import tpu_sc as plsc

_CUTOFF_ENERGY = -0.5
_CUTOFF_D2 = 3.6 * 3.6
_CUTOFF_GAP = 3
_EPS = 0.001
_COEF = 0.42 * 0.2 * 332
_LEN_NH = 1.015

_NC = 2
_NS = 16
_L = 16


def _prep_body(xt_ref, cpa_ref, c_ref, t_ref):
    n_res = xt_ref.shape[-1]
    xt = xt_ref[0]
    cpa = cpa_ref[0]
    cf = c_ref[0]

    n_at = xt[0:3]
    ca = xt[3:6]
    c_at = xt[6:9]
    o_at = xt[9:12]

    def normed(v):
        return v * lax.rsqrt(jnp.sum(v * v, axis=0, keepdims=True) + 1e-5)

    u1 = normed(n_at - cpa)
    u2 = normed(n_at - ca)
    u = normed(u1 + u2)
    h = n_at + _LEN_NH * u

    t_ref[0] = jnp.concatenate(
        [c_at, o_at, cf, jnp.zeros((1, n_res), jnp.float32),
         n_at, h, jnp.zeros((2, n_res), jnp.float32)],
        axis=0,
    )


def _rsq(x, iters=2):
    i = lax.bitcast_convert_type(x, jnp.int32)
    y = lax.bitcast_convert_type(
        jnp.int32(0x5F3759DF) - lax.shift_right_logical(i, 1), jnp.float32
    )
    hx = 0.5 * x
    for _ in range(iters):
        y = y * (1.5 - hx * y * y)
    return y


def _make_sc_kernel(B, N, K, res_per_w, chunk):
    mesh = plsc.VectorSubcoreMesh(core_axis_name="c", subcore_axis_name="s")
    w_per_b = N // res_per_w
    n_chunks = res_per_w // chunk
    ce = chunk * K

    @functools.partial(
        pl.kernel,
        out_type=(
            jax.ShapeDtypeStruct((B, N * K), jnp.float32),
            jax.ShapeDtypeStruct((B, N * K), jnp.float32),
        ),
        mesh=mesh,
        scratch_types=[
            pltpu.VMEM((16 * N,), jnp.float32),
            pltpu.VMEM((ce,), jnp.int32),
            pltpu.VMEM((ce,), jnp.int32),
            pltpu.VMEM((ce,), jnp.float32),
            pltpu.VMEM((ce,), jnp.float32),
            pltpu.VMEM((ce,), jnp.float32),
            pltpu.VMEM((ce,), jnp.float32),
            pltpu.VMEM((ce,), jnp.float32),
            pltpu.VMEM((ce,), jnp.float32),
            pltpu.SemaphoreType.DMA,
            pltpu.SemaphoreType.DMA,
            pltpu.SemaphoreType.DMA,
            pltpu.SemaphoreType.DMA,
            pltpu.SemaphoreType.DMA,
        ],
        compiler_params=pltpu.CompilerParams(needs_layout_passes=False),
    )
    def sc_kernel(t_hbm, e_hbm, m_hbm, hb_hbm, mhb_hbm,
                  tv, ev_a, ev_b, mv_a, mv_b, hbv_a, hbv_b, mhbv_a, mhbv_b,
                  sem_t, si0, si1, so0, so1):
        wid = lax.axis_index("s") * _NC + lax.axis_index("c")
        b = wid // w_per_b
        r0 = (wid % w_per_b) * res_per_w
        evs = (ev_a, ev_b)
        mvs = (mv_a, mv_b)
        hbvs = (hbv_a, hbv_b)
        mhbvs = (mhbv_a, mhbv_b)
        sem_in = (si0, si1)
        sem_out = (so0, so1)
        tcopy = pltpu.async_copy(t_hbm.at[b], tv, sem_t)

        def start_in(ci_, buf):
            e0 = (r0 + ci_ * chunk) * K
            return (
                pltpu.async_copy(
                    e_hbm.at[b, pl.ds(e0, ce)], evs[buf], sem_in[buf]
                ),
                pltpu.async_copy(
                    m_hbm.at[b, pl.ds(e0, ce)], mvs[buf], sem_in[buf]
                ),
            )

        pending_in = start_in(0, 0)
        tcopy.wait()

        def tcol(comp, idx):
            return plsc.load_gather(
                tv, [idx + jnp.full((_L,), comp * N, jnp.int32)]
            )

        pending_out = [None, None]
        for ci_ in range(n_chunks):
            buf = ci_ % 2
            c0 = r0 + ci_ * chunk
            e0 = c0 * K
            for h in pending_in:
                h.wait()
            if ci_ + 1 < n_chunks:
                pending_in = start_in(ci_ + 1, 1 - buf)
            if pending_out[buf] is not None:
                for h in pending_out[buf]:
                    h.wait()
            ev = evs[buf]
            mv = mvs[buf]
            hbv = hbvs[buf]
            mhbv = mhbvs[buf]

            @plsc.parallel_loop(0, chunk, step=1, unroll=3)
            def body(ri):
                ig = (c0 + ri).astype(jnp.int32)
                igv = jnp.full((_L,), ig, jnp.int32)
                nx = tcol(8, igv)
                ny = tcol(9, igv)
                nz = tcol(10, igv)
                hx_ = tcol(11, igv)
                hy_ = tcol(12, igv)
                hz_ = tcol(13, igv)
                chi = tcol(6, igv)
                for v in range(K // _L):
                    off = ri * K + v * _L
                    jv = ev[pl.ds(off, _L)]
                    cjx = tcol(0, jv)
                    cjy = tcol(1, jv)
                    cjz = tcol(2, jv)
                    ojx = tcol(3, jv)
                    ojy = tcol(4, jv)
                    ojz = tcol(5, jv)
                    chj = tcol(6, jv)

                    def d2(ax, ay, az, bx, by, bz):
                        dx = ax - bx
                        dy = ay - by
                        dz = az - bz
                        return dx * dx + dy * dy + dz * dz + _EPS

                    d_no = d2(nx, ny, nz, ojx, ojy, ojz)
                    d_nc = d2(nx, ny, nz, cjx, cjy, cjz)
                    d_hc = d2(hx_, hy_, hz_, cjx, cjy, cjz)
                    d_ho = d2(hx_, hy_, hz_, ojx, ojy, ojz)
                    u_ij = _COEF * (
                        _rsq(d_no) - _rsq(d_nc) + _rsq(d_hc) - _rsq(d_ho)
                    )

                    dj = jv - igv
                    is_local = (jnp.abs(dj) < _CUTOFF_GAP) & (chi == chj)
                    ok = (
                        (~is_local)
                        & (d_no < _CUTOFF_D2)
                        & (chi > 0.0)
                        & (chj > 0.0)
                    )
                    mval = mv[pl.ds(off, _L)]
                    mhb = jnp.where(ok, mval, 0.0)
                    hb = jnp.where(ok & (u_ij < _CUTOFF_ENERGY), mval, 0.0)
                    mhbv[pl.ds(off, _L)] = mhb
                    hbv[pl.ds(off, _L)] = hb

            pending_out[buf] = (
                pltpu.async_copy(hbv, hb_hbm.at[b, pl.ds(e0, ce)], sem_out[buf]),
                pltpu.async_copy(mhbv, mhb_hbm.at[b, pl.ds(e0, ce)], sem_out[buf]),
            )

        for hs in pending_out:
            if hs is not None:
                for h in hs:
                    h.wait()

    return sc_kernel


@jax.jit
def kernel(X, C, edge_idx, mask_ij):
    B, N = X.shape[0], X.shape[1]
    K = edge_idx.shape[-1]

    xt = X.reshape(B, N, 12).transpose(0, 2, 1)
    cpa = jnp.concatenate([xt[:, 6:9, :1], xt[:, 6:9, :-1]], axis=2)
    cf = C.astype(jnp.float32).reshape(B, 1, N)
    e32 = edge_idx.astype(jnp.int32).reshape(B, N * K)
    m2 = mask_ij.reshape(B, N * K)

    table = pl.pallas_call(
        _prep_body,
        grid=(B,),
        in_specs=[
            pl.BlockSpec((1, 12, N), lambda i: (i, 0, 0)),
            pl.BlockSpec((1, 3, N), lambda i: (i, 0, 0)),
            pl.BlockSpec((1, 1, N), lambda i: (i, 0, 0)),
        ],
        out_specs=pl.BlockSpec((1, 16, N), lambda i: (i, 0, 0)),
        out_shape=jax.ShapeDtypeStruct((B, 16, N), jnp.float32),
    )(xt, cpa, cf)

    n_workers = _NC * _NS
    res_per_w = (B * N) // n_workers
    chunk = min(res_per_w, 128)
    hb_flat, mhb_flat = _make_sc_kernel(B, N, K, res_per_w, chunk)(
        table.reshape(B, 16 * N), e32, m2
    )

    hbonds = hb_flat.reshape(B, N, K)
    mask_hb = mhb_flat.reshape(B, N, K)
    h_i = table[:, 11:14, :].transpose(0, 2, 1)[:, :, None, :]
    return (hbonds, mask_hb, h_i)


# speedup vs baseline: 1.0184x; 1.0184x over previous
import functools

import jax
import jax.numpy as jnp
from jax import lax
from jax.experimental import pallas as pl
from jax.experimental.pallas import tpu as pltpu
from jax.experimental.pallas import tpu_sc as plsc

_CUTOFF_ENERGY = -0.5
_CUTOFF_D2 = 3.6 * 3.6
_CUTOFF_GAP = 3
_EPS = 0.001
_COEF = 0.42 * 0.2 * 332
_LEN_NH = 1.015

_NC = 2
_NS = 16
_L = 16


def _prep_body(xt_ref, cpa_ref, c_ref, t_ref):
    n_res = xt_ref.shape[-1]
    xt = xt_ref[0]
    cpa = cpa_ref[0]
    cf = c_ref[0]

    n_at = xt[0:3]
    ca = xt[3:6]
    c_at = xt[6:9]
    o_at = xt[9:12]

    def normed(v):
        return v * lax.rsqrt(jnp.sum(v * v, axis=0, keepdims=True) + 1e-5)

    u1 = normed(n_at - cpa)
    u2 = normed(n_at - ca)
    u = normed(u1 + u2)
    h = n_at + _LEN_NH * u

    def sq(v):
        return jnp.sum(v * v, axis=0, keepdims=True)

    t_ref[0] = jnp.concatenate(
        [c_at, o_at, cf, sq(c_at), sq(o_at),
         n_at, h, sq(n_at) + _EPS, sq(h) + _EPS],
        axis=0,
    )


def _rsq(x, iters=2):
    i = lax.bitcast_convert_type(x, jnp.int32)
    y = lax.bitcast_convert_type(
        jnp.int32(0x5F3759DF) - lax.shift_right_logical(i, 1), jnp.float32
    )
    hx = 0.5 * x
    for _ in range(iters):
        y = y * (1.5 - hx * y * y)
    return y


def _make_sc_kernel(B, N, K, res_per_w, chunk):
    mesh = plsc.VectorSubcoreMesh(core_axis_name="c", subcore_axis_name="s")
    w_per_b = N // res_per_w
    n_chunks = res_per_w // chunk
    ce = chunk * K

    @functools.partial(
        pl.kernel,
        out_type=(
            jax.ShapeDtypeStruct((B, N * K), jnp.float32),
            jax.ShapeDtypeStruct((B, N * K), jnp.float32),
        ),
        mesh=mesh,
        scratch_types=[
            pltpu.VMEM((17 * N,), jnp.float32),
            pltpu.VMEM((ce,), jnp.int32),
            pltpu.VMEM((ce,), jnp.int32),
            pltpu.VMEM((ce,), jnp.float32),
            pltpu.VMEM((ce,), jnp.float32),
            pltpu.VMEM((ce,), jnp.float32),
            pltpu.VMEM((ce,), jnp.float32),
            pltpu.VMEM((ce,), jnp.float32),
            pltpu.VMEM((ce,), jnp.float32),
            pltpu.SemaphoreType.DMA,
            pltpu.SemaphoreType.DMA,
            pltpu.SemaphoreType.DMA,
            pltpu.SemaphoreType.DMA,
            pltpu.SemaphoreType.DMA,
        ],
        compiler_params=pltpu.CompilerParams(needs_layout_passes=False),
    )
    def sc_kernel(t_hbm, e_hbm, m_hbm, hb_hbm, mhb_hbm,
                  tv, ev_a, ev_b, mv_a, mv_b, hbv_a, hbv_b, mhbv_a, mhbv_b,
                  sem_t, si0, si1, so0, so1):
        wid = lax.axis_index("s") * _NC + lax.axis_index("c")
        b = wid // w_per_b
        r0 = (wid % w_per_b) * res_per_w
        evs = (ev_a, ev_b)
        mvs = (mv_a, mv_b)
        hbvs = (hbv_a, hbv_b)
        mhbvs = (mhbv_a, mhbv_b)
        sem_in = (si0, si1)
        sem_out = (so0, so1)
        tcopy = pltpu.async_copy(t_hbm.at[b], tv, sem_t)

        def start_in(ci_, buf):
            e0 = (r0 + ci_ * chunk) * K
            return (
                pltpu.async_copy(
                    e_hbm.at[b, pl.ds(e0, ce)], evs[buf], sem_in[buf]
                ),
                pltpu.async_copy(
                    m_hbm.at[b, pl.ds(e0, ce)], mvs[buf], sem_in[buf]
                ),
            )

        pending_in = start_in(0, 0)
        tcopy.wait()

        def tcol(comp, idx):
            return plsc.load_gather(
                tv, [idx + jnp.full((_L,), comp * N, jnp.int32)]
            )

        pending_out = [None, None]
        for ci_ in range(n_chunks):
            buf = ci_ % 2
            c0 = r0 + ci_ * chunk
            e0 = c0 * K
            for h in pending_in:
                h.wait()
            if ci_ + 1 < n_chunks:
                pending_in = start_in(ci_ + 1, 1 - buf)
            if pending_out[buf] is not None:
                for h in pending_out[buf]:
                    h.wait()
            ev = evs[buf]
            mv = mvs[buf]
            hbv = hbvs[buf]
            mhbv = mhbvs[buf]

            @plsc.parallel_loop(0, chunk, step=1, unroll=2)
            def body(ri):
                ig = (c0 + ri).astype(jnp.int32)
                igv = jnp.full((_L,), ig, jnp.int32)
                nx = tcol(9, igv)
                ny = tcol(10, igv)
                nz = tcol(11, igv)
                hx_ = tcol(12, igv)
                hy_ = tcol(13, igv)
                hz_ = tcol(14, igv)
                chi = tcol(6, igv)
                a_n = tcol(15, igv)
                a_h = tcol(16, igv)
                for v in range(K // _L):
                    off = ri * K + v * _L
                    jv = ev[pl.ds(off, _L)]
                    cjx = tcol(0, jv)
                    cjy = tcol(1, jv)
                    cjz = tcol(2, jv)
                    ojx = tcol(3, jv)
                    ojy = tcol(4, jv)
                    ojz = tcol(5, jv)
                    chj = tcol(6, jv)
                    c2 = tcol(7, jv)
                    o2 = tcol(8, jv)

                    def dd(px, py, pz, a_p, qx, qy, qz, q2):
                        t = px * qx + py * qy + pz * qz
                        return (a_p + q2) - (t + t)

                    d_no = dd(nx, ny, nz, a_n, ojx, ojy, ojz, o2)
                    d_nc = dd(nx, ny, nz, a_n, cjx, cjy, cjz, c2)
                    d_hc = dd(hx_, hy_, hz_, a_h, cjx, cjy, cjz, c2)
                    d_ho = dd(hx_, hy_, hz_, a_h, ojx, ojy, ojz, o2)
                    u_ij = _COEF * (
                        _rsq(d_no) - _rsq(d_nc) + _rsq(d_hc) - _rsq(d_ho)
                    )

                    dj = jv - igv
                    is_local = (jnp.abs(dj) < _CUTOFF_GAP) & (chi == chj)
                    ok = (
                        (~is_local)
                        & (d_no < _CUTOFF_D2)
                        & (chi > 0.0)
                        & (chj > 0.0)
                    )
                    mval = mv[pl.ds(off, _L)]
                    mhb = jnp.where(ok, mval, 0.0)
                    hb = jnp.where(ok & (u_ij < _CUTOFF_ENERGY), mval, 0.0)
                    mhbv[pl.ds(off, _L)] = mhb
                    hbv[pl.ds(off, _L)] = hb

            pending_out[buf] = (
                pltpu.async_copy(hbv, hb_hbm.at[b, pl.ds(e0, ce)], sem_out[buf]),
                pltpu.async_copy(mhbv, mhb_hbm.at[b, pl.ds(e0, ce)], sem_out[buf]),
            )

        for hs in pending_out:
            if hs is not None:
                for h in hs:
                    h.wait()

    return sc_kernel


@jax.jit
def kernel(X, C, edge_idx, mask_ij):
    B, N = X.shape[0], X.shape[1]
    K = edge_idx.shape[-1]

    xt = X.reshape(B, N, 12).transpose(0, 2, 1)
    cpa = jnp.concatenate([xt[:, 6:9, :1], xt[:, 6:9, :-1]], axis=2)
    cf = C.astype(jnp.float32).reshape(B, 1, N)
    e32 = edge_idx.astype(jnp.int32).reshape(B, N * K)
    m2 = mask_ij.reshape(B, N * K)

    table = pl.pallas_call(
        _prep_body,
        grid=(B,),
        in_specs=[
            pl.BlockSpec((1, 12, N), lambda i: (i, 0, 0)),
            pl.BlockSpec((1, 3, N), lambda i: (i, 0, 0)),
            pl.BlockSpec((1, 1, N), lambda i: (i, 0, 0)),
        ],
        out_specs=pl.BlockSpec((1, 17, N), lambda i: (i, 0, 0)),
        out_shape=jax.ShapeDtypeStruct((B, 17, N), jnp.float32),
    )(xt, cpa, cf)

    n_workers = _NC * _NS
    res_per_w = (B * N) // n_workers
    chunk = min(res_per_w, 128)
    hb_flat, mhb_flat = _make_sc_kernel(B, N, K, res_per_w, chunk)(
        table.reshape(B, 17 * N), e32, m2
    )

    hbonds = hb_flat.reshape(B, N, K)
    mask_hb = mhb_flat.reshape(B, N, K)
    h_i = table[:, 12:15, :].transpose(0, 2, 1)[:, :, None, :]
    return (hbonds, mask_hb, h_i)


# speedup vs baseline: 1.0516x; 1.0326x over previous
import functools

import jax
import jax.numpy as jnp
from jax import lax
from jax.experimental import pallas as pl
from jax.experimental.pallas import tpu as pltpu
from jax.experimental.pallas import tpu_sc as plsc

_CUTOFF_ENERGY = -0.5
_CUTOFF_D2 = 3.6 * 3.6
_CUTOFF_GAP = 3
_EPS = 0.001
_COEF = 0.42 * 0.2 * 332
_LEN_NH = 1.015

_NC = 2
_NS = 16
_L = 16


def _prep_body(xt_ref, cpa_ref, c_ref, t_ref):
    n_res = xt_ref.shape[-1]
    xt = xt_ref[0]
    cpa = cpa_ref[0]
    cf = c_ref[0]

    n_at = xt[0:3]
    ca = xt[3:6]
    c_at = xt[6:9]
    o_at = xt[9:12]

    def normed(v):
        return v * lax.rsqrt(jnp.sum(v * v, axis=0, keepdims=True) + 1e-5)

    u1 = normed(n_at - cpa)
    u2 = normed(n_at - ca)
    u = normed(u1 + u2)
    h = n_at + _LEN_NH * u

    t_ref[0] = jnp.concatenate(
        [c_at, o_at, cf, jnp.zeros((1, n_res), jnp.float32),
         n_at, h, jnp.zeros((2, n_res), jnp.float32)],
        axis=0,
    )


def _rsq(x, iters=2):
    i = lax.bitcast_convert_type(x, jnp.int32)
    y = lax.bitcast_convert_type(
        jnp.int32(0x5F3759DF) - lax.shift_right_logical(i, 1), jnp.float32
    )
    hx = 0.5 * x
    for _ in range(iters):
        y = y * (1.5 - hx * y * y)
    return y


def _make_sc_kernel(B, N, K, res_per_w, chunk):
    mesh = plsc.VectorSubcoreMesh(core_axis_name="c", subcore_axis_name="s")
    w_per_b = N // res_per_w
    n_chunks = res_per_w // chunk
    ce = chunk * K

    @functools.partial(
        pl.kernel,
        out_type=(
            jax.ShapeDtypeStruct((B, N * K), jnp.float32),
            jax.ShapeDtypeStruct((B, N * K), jnp.float32),
        ),
        mesh=mesh,
        scratch_types=[
            pltpu.VMEM((16 * N,), jnp.float32),
            pltpu.VMEM((ce,), jnp.int32),
            pltpu.VMEM((ce,), jnp.int32),
            pltpu.VMEM((ce,), jnp.float32),
            pltpu.VMEM((ce,), jnp.float32),
            pltpu.VMEM((ce,), jnp.float32),
            pltpu.VMEM((ce,), jnp.float32),
            pltpu.VMEM((ce,), jnp.float32),
            pltpu.VMEM((ce,), jnp.float32),
            pltpu.SemaphoreType.DMA,
            pltpu.SemaphoreType.DMA,
            pltpu.SemaphoreType.DMA,
            pltpu.SemaphoreType.DMA,
            pltpu.SemaphoreType.DMA,
        ],
        compiler_params=pltpu.CompilerParams(needs_layout_passes=False),
    )
    def sc_kernel(t_hbm, e_hbm, m_hbm, hb_hbm, mhb_hbm,
                  tv, ev_a, ev_b, mv_a, mv_b, hbv_a, hbv_b, mhbv_a, mhbv_b,
                  sem_t, si0, si1, so0, so1):
        wid = lax.axis_index("s") * _NC + lax.axis_index("c")
        b = wid // w_per_b
        r0 = (wid % w_per_b) * res_per_w
        evs = (ev_a, ev_b)
        mvs = (mv_a, mv_b)
        hbvs = (hbv_a, hbv_b)
        mhbvs = (mhbv_a, mhbv_b)
        sem_in = (si0, si1)
        sem_out = (so0, so1)
        tcopy = pltpu.async_copy(t_hbm.at[b], tv, sem_t)

        def start_in(ci_, buf):
            e0 = (r0 + ci_ * chunk) * K
            return (
                pltpu.async_copy(
                    e_hbm.at[b, pl.ds(e0, ce)], evs[buf], sem_in[buf]
                ),
                pltpu.async_copy(
                    m_hbm.at[b, pl.ds(e0, ce)], mvs[buf], sem_in[buf]
                ),
            )

        pending_in = start_in(0, 0)
        tcopy.wait()

        def tcol(comp, idx):
            return plsc.load_gather(
                tv, [idx + jnp.full((_L,), comp * N, jnp.int32)]
            )

        pending_out = [None, None]
        for ci_ in range(n_chunks):
            buf = ci_ % 2
            c0 = r0 + ci_ * chunk
            e0 = c0 * K
            for h in pending_in:
                h.wait()
            if ci_ + 1 < n_chunks:
                pending_in = start_in(ci_ + 1, 1 - buf)
            if pending_out[buf] is not None:
                for h in pending_out[buf]:
                    h.wait()
            ev = evs[buf]
            mv = mvs[buf]
            hbv = hbvs[buf]
            mhbv = mhbvs[buf]

            @plsc.parallel_loop(0, chunk, step=1, unroll=2)
            def body(ri):
                ig = (c0 + ri).astype(jnp.int32)
                igv = jnp.full((_L,), ig, jnp.int32)
                nx = tcol(8, igv)
                ny = tcol(9, igv)
                nz = tcol(10, igv)
                hx_ = tcol(11, igv)
                hy_ = tcol(12, igv)
                hz_ = tcol(13, igv)
                chi = tcol(6, igv)
                for v in range(K // _L):
                    off = ri * K + v * _L
                    jv = ev[pl.ds(off, _L)]
                    cjx = tcol(0, jv)
                    cjy = tcol(1, jv)
                    cjz = tcol(2, jv)
                    ojx = tcol(3, jv)
                    ojy = tcol(4, jv)
                    ojz = tcol(5, jv)
                    chj = tcol(6, jv)

                    def d2(ax, ay, az, bx, by, bz):
                        dx = ax - bx
                        dy = ay - by
                        dz = az - bz
                        return dx * dx + dy * dy + dz * dz + _EPS

                    d_no = d2(nx, ny, nz, ojx, ojy, ojz)
                    d_nc = d2(nx, ny, nz, cjx, cjy, cjz)
                    d_hc = d2(hx_, hy_, hz_, cjx, cjy, cjz)
                    d_ho = d2(hx_, hy_, hz_, ojx, ojy, ojz)
                    u_ij = _COEF * (
                        _rsq(d_no) - _rsq(d_nc) + _rsq(d_hc) - _rsq(d_ho)
                    )

                    dj = jv - igv
                    is_local = (jnp.abs(dj) < _CUTOFF_GAP) & (chi == chj)
                    ok = (
                        (~is_local)
                        & (d_no < _CUTOFF_D2)
                        & (chi > 0.0)
                        & (chj > 0.0)
                    )
                    mval = mv[pl.ds(off, _L)]
                    mhb = jnp.where(ok, mval, 0.0)
                    hb = jnp.where(ok & (u_ij < _CUTOFF_ENERGY), mval, 0.0)
                    mhbv[pl.ds(off, _L)] = mhb
                    hbv[pl.ds(off, _L)] = hb

            pending_out[buf] = (
                pltpu.async_copy(hbv, hb_hbm.at[b, pl.ds(e0, ce)], sem_out[buf]),
                pltpu.async_copy(mhbv, mhb_hbm.at[b, pl.ds(e0, ce)], sem_out[buf]),
            )

        for hs in pending_out:
            if hs is not None:
                for h in hs:
                    h.wait()

    return sc_kernel


@jax.jit
def kernel(X, C, edge_idx, mask_ij):
    B, N = X.shape[0], X.shape[1]
    K = edge_idx.shape[-1]

    xt = X.reshape(B, N, 12).transpose(0, 2, 1)
    cpa = jnp.concatenate([xt[:, 6:9, :1], xt[:, 6:9, :-1]], axis=2)
    cf = C.astype(jnp.float32).reshape(B, 1, N)
    e32 = edge_idx.astype(jnp.int32).reshape(B, N * K)
    m2 = mask_ij.reshape(B, N * K)

    table = pl.pallas_call(
        _prep_body,
        grid=(B,),
        in_specs=[
            pl.BlockSpec((1, 12, N), lambda i: (i, 0, 0)),
            pl.BlockSpec((1, 3, N), lambda i: (i, 0, 0)),
            pl.BlockSpec((1, 1, N), lambda i: (i, 0, 0)),
        ],
        out_specs=pl.BlockSpec((1, 16, N), lambda i: (i, 0, 0)),
        out_shape=jax.ShapeDtypeStruct((B, 16, N), jnp.float32),
    )(xt, cpa, cf)

    n_workers = _NC * _NS
    res_per_w = (B * N) // n_workers
    chunk = min(res_per_w, 128)
    hb_flat, mhb_flat = _make_sc_kernel(B, N, K, res_per_w, chunk)(
        table.reshape(B, 16 * N), e32, m2
    )

    hbonds = hb_flat.reshape(B, N, K)
    mask_hb = mhb_flat.reshape(B, N, K)
    h_i = table[:, 11:14, :].transpose(0, 2, 1)[:, :, None, :]
    return (hbonds, mask_hb, h_i)
